# baseline (speedup 1.0000x reference)
import functools

import jax
import jax.numpy as jnp
from jax import lax
from jax.experimental import pallas as pl
from jax.experimental.pallas import tpu as pltpu
from jax.experimental.pallas import tpu_sc as plsc

OFF_THRESH = 0.5

H = 384
W = 384
B = 2
HP = H + 2
WBUF = 416
ROWS_PER_WORKER = 24
S1_ROWS = ROWS_PER_WORKER + 2
Z_ROWS = S1_ROWS + 2
NC = 2
NS = 16

C28 = 1 << 28
E_CASE_B = -(4 << 28) + C28 - 1
E_CASE_CD = 1 << 29


def _sc_body(xp_hbm, c_hbm, out_hbm,
             zbuf, xbuf, ybuf, mbuf, ebuf, ox, oy, oz, om, cbuf):
    wid = lax.axis_index("s") * NC + lax.axis_index("c")
    b = wid // NS
    wi = wid % NS
    r0 = ROWS_PER_WORKER * wi
    ch = b * 4

    pltpu.sync_copy(c_hbm, cbuf)
    pltpu.sync_copy(xp_hbm.at[pl.ds((ch + 0) * 388 + r0 + 1, S1_ROWS), :], xbuf)
    pltpu.sync_copy(xp_hbm.at[pl.ds((ch + 1) * 388 + r0 + 1, S1_ROWS), :], ybuf)
    pltpu.sync_copy(xp_hbm.at[pl.ds((ch + 2) * 388 + r0, Z_ROWS), :], zbuf)
    pltpu.sync_copy(xp_hbm.at[pl.ds((ch + 3) * 388 + r0 + 1, S1_ROWS), :], mbuf)

    a = [cbuf[i, :] for i in range(9)]
    iota_i = lax.iota(jnp.int32, 16)
    iota = iota_i.astype(jnp.float32)

    def s1_row(rr, _):
        vf = (r0 + rr - 1).astype(jnp.float32)
        rowx = a[1] * vf + a[2]
        rowy = a[4] * vf + a[5]
        rowz = a[7] * vf + a[8]

        def s1_chunk(j, _):
            c0 = 16 * j
            bc = c0 + 1
            u = iota + (c0 - 1).astype(jnp.float32)
            rx_c = a[0] * u + rowx
            ry_c = a[3] * u + rowy
            rz_c = a[6] * u + rowz
            xc = xbuf[rr, pl.ds(bc, 16)]
            yc = ybuf[rr, pl.ds(bc, 16)]
            zc = zbuf[rr + 1, pl.ds(bc, 16)]
            mc = mbuf[rr, pl.ds(bc, 16)]
            best_d = None
            best_k = None
            for k in range(9):
                dr, dc = k // 3 - 1, k % 3 - 1
                zn = zbuf[rr + 1 + dr, pl.ds(bc + dc, 16)]
                rx = rx_c + cbuf[9 + k, :]
                ry = ry_c + cbuf[18 + k, :]
                rz = rz_c + cbuf[27 + k, :]
                dx = rx * zn - xc
                dy = ry * zn - yc
                dz = rz * zn - zc
                d = dx * dx + dy * dy + dz * dz
                if best_d is None:
                    best_d = d
                    best_k = jnp.zeros((16,), jnp.int32)
                else:
                    m = d < best_d
                    best_d = jnp.where(m, d, best_d)
                    best_k = jnp.where(m, k, best_k)
            rmask = mc > OFF_THRESH
            zmask = zc > 0.0
            zb2 = lax.bitcast_convert_type(zc, jnp.int32) >> 2
            e_a = (best_k - 8) * jnp.int32(C28) + zb2
            e = jnp.where(rmask,
                          jnp.where(zmask, e_a, jnp.int32(E_CASE_B)),
                          jnp.int32(E_CASE_CD))
            ebuf[rr, pl.ds(bc, 16)] = e
            return 0

        lax.fori_loop(0, 25, s1_chunk, 0)
        return 0

    lax.fori_loop(0, S1_ROWS, s1_row, 0)

    def s2_row(rr2, _):
        xr = rr2 + 1

        def s2_chunk(j2, _):
            b0 = 16 * j2 + 2
            e_c = ebuf[xr, pl.ds(b0, 16)]
            best = jnp.full((16,), C28, jnp.int32)
            bk = jnp.full((16,), 4, jnp.int32)
            for k in range(9):
                ro = -(k // 3 - 1)
                co = -(k % 3 - 1)
                if ro == 0 and co == 0:
                    e_q = e_c
                else:
                    e_q = ebuf[xr + ro, pl.ds(b0 + co, 16)]
                cand = e_q - jnp.int32((k - 8) * C28)
                m = (cand > 0) & (cand < best)
                best = jnp.where(m, cand, best)
                bk = jnp.where(m, k, bk)
            s = (bk * 11) >> 5
            t = bk - 3 * s
            rowq = (xr + 1) - s
            colq = iota_i + (b0 + 1) - t
            xq = plsc.load_gather(xbuf, [rowq, colq])
            yq = plsc.load_gather(ybuf, [rowq, colq])
            zq = plsc.load_gather(zbuf, [rowq + 1, colq])
            mq = plsc.load_gather(mbuf, [rowq, colq])
            found = best < C28
            t4 = e_c + jnp.int32(4 * C28)
            c4 = (t4 > 0) & (t4 < C28)
            cm4 = c4 | (e_c == jnp.int32(E_CASE_CD))
            fx = found | c4
            fm = found | cm4
            zero = jnp.zeros((16,), jnp.float32)
            ox[rr2, pl.ds(16 * j2, 16)] = jnp.where(fx, xq, zero)
            oy[rr2, pl.ds(16 * j2, 16)] = jnp.where(fx, yq, zero)
            oz[rr2, pl.ds(16 * j2, 16)] = jnp.where(fx, zq, zero)
            om[rr2, pl.ds(16 * j2, 16)] = jnp.where(fm, mq, zero)
            return 0

        lax.fori_loop(0, W // 16, s2_chunk, 0)
        return 0

    lax.fori_loop(0, ROWS_PER_WORKER, s2_row, 0)

    base = b * 4 * H + r0
    pltpu.sync_copy(ox, out_hbm.at[pl.ds(base + 0 * H, ROWS_PER_WORKER), :])
    pltpu.sync_copy(oy, out_hbm.at[pl.ds(base + 1 * H, ROWS_PER_WORKER), :])
    pltpu.sync_copy(oz, out_hbm.at[pl.ds(base + 2 * H, ROWS_PER_WORKER), :])
    pltpu.sync_copy(om, out_hbm.at[pl.ds(base + 3 * H, ROWS_PER_WORKER), :])


_smap_sc = functools.partial(
    pl.kernel,
    out_type=jax.ShapeDtypeStruct((B * 4 * H, W), jnp.float32),
    mesh=plsc.VectorSubcoreMesh(core_axis_name="c", subcore_axis_name="s"),
    compiler_params=pltpu.CompilerParams(
        use_tc_tiling_on_sc=False, needs_layout_passes=False),
    scratch_types=[
        pltpu.VMEM((Z_ROWS, WBUF), jnp.float32),
        pltpu.VMEM((S1_ROWS, WBUF), jnp.float32),
        pltpu.VMEM((S1_ROWS, WBUF), jnp.float32),
        pltpu.VMEM((S1_ROWS, WBUF), jnp.float32),
        pltpu.VMEM((S1_ROWS, WBUF), jnp.int32),
        pltpu.VMEM((ROWS_PER_WORKER, W), jnp.float32),
        pltpu.VMEM((ROWS_PER_WORKER, W), jnp.float32),
        pltpu.VMEM((ROWS_PER_WORKER, W), jnp.float32),
        pltpu.VMEM((ROWS_PER_WORKER, W), jnp.float32),
        pltpu.VMEM((36, 16), jnp.float32),
    ],
)(_sc_body)


def kernel(x, camera_matrix):
    k_inv = jnp.linalg.inv(camera_matrix)
    dc = jnp.array([k % 3 - 1 for k in range(9)], jnp.float32)
    dr = jnp.array([k // 3 - 1 for k in range(9)], jnp.float32)
    sx = k_inv[0, 0] * dc + k_inv[0, 1] * dr
    sy = k_inv[1, 0] * dc + k_inv[1, 1] * dr
    sz = k_inv[2, 0] * dc + k_inv[2, 1] * dr
    consts = jnp.concatenate([k_inv.reshape(9), sx, sy, sz])
    consts = jnp.repeat(consts.reshape(36, 1), 16, axis=1)
    xp = jnp.pad(x, ((0, 0), (0, 0), (2, 2), (2, WBUF - W - 2)))
    xp = xp.reshape(B * 4 * (HP + 2), WBUF)
    out = _smap_sc(xp, consts)
    return out.reshape(B, 4, H, W)


# speedup vs baseline: 1.1848x; 1.1848x over previous
import functools

import jax
import jax.numpy as jnp
from jax import lax
from jax.experimental import pallas as pl
from jax.experimental.pallas import tpu as pltpu
from jax.experimental.pallas import tpu_sc as plsc

OFF_THRESH = 0.5

H = 384
W = 384
B = 2
HP = H + 2
WBUF = 416
ROWS_PER_WORKER = 24
S1_ROWS = ROWS_PER_WORKER + 2
Z_ROWS = S1_ROWS + 2
NC = 2
NS = 16

C28 = 1 << 28
E_CASE_B = -(4 << 28) + C28 - 1
E_CASE_CD = 1 << 29


def _sc_body(xp_hbm, c_hbm, out_hbm,
             zbuf, xbuf, ybuf, mbuf, ebuf, ox, oy, oz, om, cbuf):
    wid = lax.axis_index("s") * NC + lax.axis_index("c")
    b = wid // NS
    wi = wid % NS
    r0 = ROWS_PER_WORKER * wi
    ch = b * 4

    pltpu.sync_copy(c_hbm, cbuf)
    pltpu.sync_copy(xp_hbm.at[pl.ds((ch + 0) * 388 + r0 + 1, S1_ROWS), :], xbuf)
    pltpu.sync_copy(xp_hbm.at[pl.ds((ch + 1) * 388 + r0 + 1, S1_ROWS), :], ybuf)
    pltpu.sync_copy(xp_hbm.at[pl.ds((ch + 2) * 388 + r0, Z_ROWS), :], zbuf)
    pltpu.sync_copy(xp_hbm.at[pl.ds((ch + 3) * 388 + r0 + 1, S1_ROWS), :], mbuf)

    a = [cbuf[i, :] for i in range(9)]
    iota_i = lax.iota(jnp.int32, 16)
    iota = iota_i.astype(jnp.float32)

    def s1_row(rr, _):
        vf = (r0 + rr - 1).astype(jnp.float32)
        rowx = a[1] * vf + a[2]
        rowy = a[4] * vf + a[5]
        rowz = a[7] * vf + a[8]

        @plsc.parallel_loop(0, 25)
        def s1_chunk(j):
            c0 = 16 * j
            bc = c0 + 1
            u = iota + (c0 - 1).astype(jnp.float32)
            rx_c = a[0] * u + rowx
            ry_c = a[3] * u + rowy
            rz_c = a[6] * u + rowz
            xc = xbuf[rr, pl.ds(bc, 16)]
            yc = ybuf[rr, pl.ds(bc, 16)]
            zc = zbuf[rr + 1, pl.ds(bc, 16)]
            mc = mbuf[rr, pl.ds(bc, 16)]
            ds = []
            for k in range(9):
                dr, dc = k // 3 - 1, k % 3 - 1
                zn = zbuf[rr + 1 + dr, pl.ds(bc + dc, 16)]
                rx = rx_c + cbuf[9 + k, :]
                ry = ry_c + cbuf[18 + k, :]
                rz = rz_c + cbuf[27 + k, :]
                dx = rx * zn - xc
                dy = ry * zn - yc
                dz = rz * zn - zc
                ds.append(dx * dx + dy * dy + dz * dz)
            pairs = [(ds[k], jnp.full((16,), k, jnp.int32)) for k in range(9)]
            while len(pairs) > 1:
                nxt = []
                for i in range(0, len(pairs) - 1, 2):
                    (da, ka), (db, kb) = pairs[i], pairs[i + 1]
                    m = db < da
                    nxt.append((jnp.where(m, db, da), jnp.where(m, kb, ka)))
                if len(pairs) % 2:
                    nxt.append(pairs[-1])
                pairs = nxt
            best_k = pairs[0][1]
            rmask = mc > OFF_THRESH
            zmask = zc > 0.0
            zb2 = lax.bitcast_convert_type(zc, jnp.int32) >> 2
            e_a = (best_k - 8) * jnp.int32(C28) + zb2
            e = jnp.where(rmask,
                          jnp.where(zmask, e_a, jnp.int32(E_CASE_B)),
                          jnp.int32(E_CASE_CD))
            ebuf[rr, pl.ds(bc, 16)] = e

        return 0

    lax.fori_loop(0, S1_ROWS, s1_row, 0)

    def s2_row(rr2, _):
        xr = rr2 + 1

        @plsc.parallel_loop(0, W // 16)
        def s2_chunk(j2):
            b0 = 16 * j2 + 2
            e_c = ebuf[xr, pl.ds(b0, 16)]
            sent = jnp.full((16,), C28, jnp.int32)
            pairs = []
            for k in range(9):
                ro = -(k // 3 - 1)
                co = -(k % 3 - 1)
                if ro == 0 and co == 0:
                    e_q = e_c
                else:
                    e_q = ebuf[xr + ro, pl.ds(b0 + co, 16)]
                cand = e_q - jnp.int32((k - 8) * C28)
                valid = (cand > 0) & (cand < sent)
                key = jnp.where(valid, cand, sent)
                pairs.append((key, jnp.full((16,), k, jnp.int32)))
            while len(pairs) > 1:
                nxt = []
                for i in range(0, len(pairs) - 1, 2):
                    (da, ka), (db, kb) = pairs[i], pairs[i + 1]
                    m = db < da
                    nxt.append((jnp.where(m, db, da), jnp.where(m, kb, ka)))
                if len(pairs) % 2:
                    nxt.append(pairs[-1])
                pairs = nxt
            best, bk = pairs[0]
            found = best < sent
            bk = jnp.where(found, bk, 4)
            s = (bk * 11) >> 5
            t = bk - 3 * s
            rowq = (xr + 1) - s
            colq = iota_i + (b0 + 1) - t
            xq = plsc.load_gather(xbuf, [rowq, colq])
            yq = plsc.load_gather(ybuf, [rowq, colq])
            zq = plsc.load_gather(zbuf, [rowq + 1, colq])
            mq = plsc.load_gather(mbuf, [rowq, colq])
            t4 = e_c + jnp.int32(4 * C28)
            c4 = (t4 > 0) & (t4 < C28)
            cm4 = c4 | (e_c == jnp.int32(E_CASE_CD))
            fx = found | c4
            fm = found | cm4
            zero = jnp.zeros((16,), jnp.float32)
            ox[rr2, pl.ds(16 * j2, 16)] = jnp.where(fx, xq, zero)
            oy[rr2, pl.ds(16 * j2, 16)] = jnp.where(fx, yq, zero)
            oz[rr2, pl.ds(16 * j2, 16)] = jnp.where(fx, zq, zero)
            om[rr2, pl.ds(16 * j2, 16)] = jnp.where(fm, mq, zero)

        return 0

    lax.fori_loop(0, ROWS_PER_WORKER, s2_row, 0)

    base = b * 4 * H + r0
    pltpu.sync_copy(ox, out_hbm.at[pl.ds(base + 0 * H, ROWS_PER_WORKER), :])
    pltpu.sync_copy(oy, out_hbm.at[pl.ds(base + 1 * H, ROWS_PER_WORKER), :])
    pltpu.sync_copy(oz, out_hbm.at[pl.ds(base + 2 * H, ROWS_PER_WORKER), :])
    pltpu.sync_copy(om, out_hbm.at[pl.ds(base + 3 * H, ROWS_PER_WORKER), :])


_smap_sc = functools.partial(
    pl.kernel,
    out_type=jax.ShapeDtypeStruct((B * 4 * H, W), jnp.float32),
    mesh=plsc.VectorSubcoreMesh(core_axis_name="c", subcore_axis_name="s"),
    compiler_params=pltpu.CompilerParams(
        use_tc_tiling_on_sc=False, needs_layout_passes=False),
    scratch_types=[
        pltpu.VMEM((Z_ROWS, WBUF), jnp.float32),
        pltpu.VMEM((S1_ROWS, WBUF), jnp.float32),
        pltpu.VMEM((S1_ROWS, WBUF), jnp.float32),
        pltpu.VMEM((S1_ROWS, WBUF), jnp.float32),
        pltpu.VMEM((S1_ROWS, WBUF), jnp.int32),
        pltpu.VMEM((ROWS_PER_WORKER, W), jnp.float32),
        pltpu.VMEM((ROWS_PER_WORKER, W), jnp.float32),
        pltpu.VMEM((ROWS_PER_WORKER, W), jnp.float32),
        pltpu.VMEM((ROWS_PER_WORKER, W), jnp.float32),
        pltpu.VMEM((36, 16), jnp.float32),
    ],
)(_sc_body)


def kernel(x, camera_matrix):
    k_inv = jnp.linalg.inv(camera_matrix)
    dc = jnp.array([k % 3 - 1 for k in range(9)], jnp.float32)
    dr = jnp.array([k // 3 - 1 for k in range(9)], jnp.float32)
    sx = k_inv[0, 0] * dc + k_inv[0, 1] * dr
    sy = k_inv[1, 0] * dc + k_inv[1, 1] * dr
    sz = k_inv[2, 0] * dc + k_inv[2, 1] * dr
    consts = jnp.concatenate([k_inv.reshape(9), sx, sy, sz])
    consts = jnp.repeat(consts.reshape(36, 1), 16, axis=1)
    xp = jnp.pad(x, ((0, 0), (0, 0), (2, 2), (2, WBUF - W - 2)))
    xp = xp.reshape(B * 4 * (HP + 2), WBUF)
    out = _smap_sc(xp, consts)
    return out.reshape(B, 4, H, W)


# speedup vs baseline: 1.2020x; 1.0146x over previous
import functools

import jax
import jax.numpy as jnp
from jax import lax
from jax.experimental import pallas as pl
from jax.experimental.pallas import tpu as pltpu
from jax.experimental.pallas import tpu_sc as plsc

OFF_THRESH = 0.5

H = 384
W = 384
B = 2
HP = H + 2
WBUF = 416
ROWS_PER_WORKER = 24
S1_ROWS = ROWS_PER_WORKER + 2
Z_ROWS = S1_ROWS + 2
NC = 2
NS = 16

C28 = 1 << 28
E_CASE_B = -(4 << 28) + C28 - 1
E_CASE_CD = 1 << 29


def _sc_body(xp_hbm, c_hbm, out_hbm,
             zbuf, xbuf, ybuf, mbuf, ebuf, ox, oy, oz, om, cbuf):
    wid = lax.axis_index("s") * NC + lax.axis_index("c")
    b = wid // NS
    wi = wid % NS
    r0 = ROWS_PER_WORKER * wi
    ch = b * 4

    pltpu.sync_copy(c_hbm, cbuf)
    pltpu.sync_copy(xp_hbm.at[pl.ds((ch + 0) * 388 + r0 + 1, S1_ROWS), :], xbuf)
    pltpu.sync_copy(xp_hbm.at[pl.ds((ch + 1) * 388 + r0 + 1, S1_ROWS), :], ybuf)
    pltpu.sync_copy(xp_hbm.at[pl.ds((ch + 2) * 388 + r0, Z_ROWS), :], zbuf)
    pltpu.sync_copy(xp_hbm.at[pl.ds((ch + 3) * 388 + r0 + 1, S1_ROWS), :], mbuf)

    a = [cbuf[i, :] for i in range(9)]
    iota_i = lax.iota(jnp.int32, 16)
    iota = iota_i.astype(jnp.float32)

    def s1_row(rr, _):
        vf = (r0 + rr - 1).astype(jnp.float32)
        rowx = a[1] * vf + a[2]
        rowy = a[4] * vf + a[5]
        rowz = a[7] * vf + a[8]

        @plsc.parallel_loop(0, 25, unroll=2)
        def s1_chunk(j):
            c0 = 16 * j
            bc = c0 + 1
            u = iota + (c0 - 1).astype(jnp.float32)
            rx_c = a[0] * u + rowx
            ry_c = a[3] * u + rowy
            rz_c = a[6] * u + rowz
            xc = xbuf[rr, pl.ds(bc, 16)]
            yc = ybuf[rr, pl.ds(bc, 16)]
            zc = zbuf[rr + 1, pl.ds(bc, 16)]
            mc = mbuf[rr, pl.ds(bc, 16)]
            ds = []
            for k in range(9):
                dr, dc = k // 3 - 1, k % 3 - 1
                zn = zbuf[rr + 1 + dr, pl.ds(bc + dc, 16)]
                rx = rx_c + cbuf[9 + k, :]
                ry = ry_c + cbuf[18 + k, :]
                rz = rz_c + cbuf[27 + k, :]
                dx = rx * zn - xc
                dy = ry * zn - yc
                dz = rz * zn - zc
                ds.append(dx * dx + dy * dy + dz * dz)
            pairs = [(ds[k], jnp.full((16,), k, jnp.int32)) for k in range(9)]
            while len(pairs) > 1:
                nxt = []
                for i in range(0, len(pairs) - 1, 2):
                    (da, ka), (db, kb) = pairs[i], pairs[i + 1]
                    m = db < da
                    nxt.append((jnp.where(m, db, da), jnp.where(m, kb, ka)))
                if len(pairs) % 2:
                    nxt.append(pairs[-1])
                pairs = nxt
            best_k = pairs[0][1]
            rmask = mc > OFF_THRESH
            zmask = zc > 0.0
            zb2 = lax.bitcast_convert_type(zc, jnp.int32) >> 2
            e_a = (best_k - 8) * jnp.int32(C28) + zb2
            e = jnp.where(rmask,
                          jnp.where(zmask, e_a, jnp.int32(E_CASE_B)),
                          jnp.int32(E_CASE_CD))
            ebuf[rr, pl.ds(bc, 16)] = e

        return 0

    lax.fori_loop(0, S1_ROWS, s1_row, 0)

    def s2_row(rr2, _):
        xr = rr2 + 1

        @plsc.parallel_loop(0, W // 16, unroll=2)
        def s2_chunk(j2):
            b0 = 16 * j2 + 2
            e_c = ebuf[xr, pl.ds(b0, 16)]
            sent = jnp.full((16,), C28, jnp.int32)
            pairs = []
            for k in range(9):
                ro = -(k // 3 - 1)
                co = -(k % 3 - 1)
                if ro == 0 and co == 0:
                    e_q = e_c
                else:
                    e_q = ebuf[xr + ro, pl.ds(b0 + co, 16)]
                cand = e_q - jnp.int32((k - 8) * C28)
                valid = (cand > 0) & (cand < sent)
                key = jnp.where(valid, cand, sent)
                pairs.append((key, jnp.full((16,), k, jnp.int32)))
            while len(pairs) > 1:
                nxt = []
                for i in range(0, len(pairs) - 1, 2):
                    (da, ka), (db, kb) = pairs[i], pairs[i + 1]
                    m = db < da
                    nxt.append((jnp.where(m, db, da), jnp.where(m, kb, ka)))
                if len(pairs) % 2:
                    nxt.append(pairs[-1])
                pairs = nxt
            best, bk = pairs[0]
            found = best < sent
            bk = jnp.where(found, bk, 4)
            s = (bk * 11) >> 5
            t = bk - 3 * s
            rowq = (xr + 1) - s
            colq = iota_i + (b0 + 1) - t
            xq = plsc.load_gather(xbuf, [rowq, colq])
            yq = plsc.load_gather(ybuf, [rowq, colq])
            zq = plsc.load_gather(zbuf, [rowq + 1, colq])
            mq = plsc.load_gather(mbuf, [rowq, colq])
            t4 = e_c + jnp.int32(4 * C28)
            c4 = (t4 > 0) & (t4 < C28)
            cm4 = c4 | (e_c == jnp.int32(E_CASE_CD))
            fx = found | c4
            fm = found | cm4
            zero = jnp.zeros((16,), jnp.float32)
            ox[rr2, pl.ds(16 * j2, 16)] = jnp.where(fx, xq, zero)
            oy[rr2, pl.ds(16 * j2, 16)] = jnp.where(fx, yq, zero)
            oz[rr2, pl.ds(16 * j2, 16)] = jnp.where(fx, zq, zero)
            om[rr2, pl.ds(16 * j2, 16)] = jnp.where(fm, mq, zero)

        return 0

    lax.fori_loop(0, ROWS_PER_WORKER, s2_row, 0)

    base = b * 4 * H + r0
    pltpu.sync_copy(ox, out_hbm.at[pl.ds(base + 0 * H, ROWS_PER_WORKER), :])
    pltpu.sync_copy(oy, out_hbm.at[pl.ds(base + 1 * H, ROWS_PER_WORKER), :])
    pltpu.sync_copy(oz, out_hbm.at[pl.ds(base + 2 * H, ROWS_PER_WORKER), :])
    pltpu.sync_copy(om, out_hbm.at[pl.ds(base + 3 * H, ROWS_PER_WORKER), :])


_smap_sc = functools.partial(
    pl.kernel,
    out_type=jax.ShapeDtypeStruct((B * 4 * H, W), jnp.float32),
    mesh=plsc.VectorSubcoreMesh(core_axis_name="c", subcore_axis_name="s"),
    compiler_params=pltpu.CompilerParams(
        use_tc_tiling_on_sc=False, needs_layout_passes=False),
    scratch_types=[
        pltpu.VMEM((Z_ROWS, WBUF), jnp.float32),
        pltpu.VMEM((S1_ROWS, WBUF), jnp.float32),
        pltpu.VMEM((S1_ROWS, WBUF), jnp.float32),
        pltpu.VMEM((S1_ROWS, WBUF), jnp.float32),
        pltpu.VMEM((S1_ROWS, WBUF), jnp.int32),
        pltpu.VMEM((ROWS_PER_WORKER, W), jnp.float32),
        pltpu.VMEM((ROWS_PER_WORKER, W), jnp.float32),
        pltpu.VMEM((ROWS_PER_WORKER, W), jnp.float32),
        pltpu.VMEM((ROWS_PER_WORKER, W), jnp.float32),
        pltpu.VMEM((36, 16), jnp.float32),
    ],
)(_sc_body)


def kernel(x, camera_matrix):
    k_inv = jnp.linalg.inv(camera_matrix)
    dc = jnp.array([k % 3 - 1 for k in range(9)], jnp.float32)
    dr = jnp.array([k // 3 - 1 for k in range(9)], jnp.float32)
    sx = k_inv[0, 0] * dc + k_inv[0, 1] * dr
    sy = k_inv[1, 0] * dc + k_inv[1, 1] * dr
    sz = k_inv[2, 0] * dc + k_inv[2, 1] * dr
    consts = jnp.concatenate([k_inv.reshape(9), sx, sy, sz])
    consts = jnp.repeat(consts.reshape(36, 1), 16, axis=1)
    xp = jnp.pad(x, ((0, 0), (0, 0), (2, 2), (2, WBUF - W - 2)))
    xp = xp.reshape(B * 4 * (HP + 2), WBUF)
    out = _smap_sc(xp, consts)
    return out.reshape(B, 4, H, W)
